# baseline (speedup 1.0000x reference)
import jax
import jax.numpy as jnp
from jax.experimental import pallas as pl
from jax.experimental.pallas import tpu as pltpu

NUM_FEATURES = 2048
LANES = 128
NT = NUM_FEATURES // LANES
BT = 512


def _transpose_block(x_ref, o_ref):
    o_ref[...] = x_ref[...].T


def _gather_block(idx_ref, x_ref, z_ref, scratch_ref):
    k = pl.program_id(1)
    scratch_ref[pl.ds(k, 1), :] = x_ref[0, :, :]

    @pl.when(k == LANES - 1)
    def _():
        z_ref[...] = scratch_ref[...].T


def kernel(x, idx):
    n, f = x.shape

    xt = pl.pallas_call(
        _transpose_block,
        grid=(f // BT, n // BT),
        in_specs=[pl.BlockSpec((BT, BT), lambda i, j: (j, i))],
        out_specs=pl.BlockSpec((BT, BT), lambda i, j: (i, j)),
        out_shape=jax.ShapeDtypeStruct((f, n), x.dtype),
    )(x)

    xt3 = xt.reshape(f, 1, n)
    z = pl.pallas_call(
        _gather_block,
        grid_spec=pltpu.PrefetchScalarGridSpec(
            num_scalar_prefetch=1,
            grid=(NT, LANES),
            in_specs=[
                pl.BlockSpec(
                    (1, 1, n),
                    lambda t, k, idx_ref: (idx_ref[t * LANES + k], 0, 0)),
            ],
            out_specs=pl.BlockSpec((n, LANES), lambda t, k, idx_ref: (0, t)),
            scratch_shapes=[pltpu.VMEM((LANES, n), x.dtype)],
        ),
        out_shape=jax.ShapeDtypeStruct((n, f), x.dtype),
    )(idx, xt3)

    logdet = jnp.zeros((n,), dtype=x.dtype)
    return (z, logdet)


# speedup vs baseline: 1.8570x; 1.8570x over previous
import jax
import jax.numpy as jnp
from jax.experimental import pallas as pl

NUM_FEATURES = 2048
LANES = 128
NT = NUM_FEATURES // LANES
BR = 64
SLICE = 8


def _permute_block(x_ref, idx_ref, z_ref):
    idx = idx_ref[0:1, :]
    lane_t = []
    src_t = []
    for t in range(NT):
        it = idx[:, t * LANES:(t + 1) * LANES]
        lane_t.append(jnp.broadcast_to(it % LANES, (SLICE, LANES)))
        src_t.append(jnp.broadcast_to(it // LANES, (SLICE, LANES)))

    for r0 in range(0, BR, SLICE):
        xs = x_ref[r0:r0 + SLICE, :]
        srcs = [xs[:, s * LANES:(s + 1) * LANES] for s in range(NT)]
        for t in range(NT):
            acc = None
            for s in range(NT):
                g = jnp.take_along_axis(srcs[s], lane_t[t], axis=1)
                m = src_t[t] == s
                acc = jnp.where(m, g, 0.0) if acc is None else jnp.where(m, g, acc)
            z_ref[r0:r0 + SLICE, t * LANES:(t + 1) * LANES] = acc


def kernel(x, idx):
    n, f = x.shape
    idx2d = idx.reshape(1, f)
    z = pl.pallas_call(
        _permute_block,
        grid=(n // BR,),
        in_specs=[
            pl.BlockSpec((BR, f), lambda i: (i, 0)),
            pl.BlockSpec((1, f), lambda i: (0, 0)),
        ],
        out_specs=pl.BlockSpec((BR, f), lambda i: (i, 0)),
        out_shape=jax.ShapeDtypeStruct((n, f), x.dtype),
    )(x, idx2d)
    logdet = jnp.zeros((n,), dtype=x.dtype)
    return (z, logdet)
